# baseline (speedup 1.0000x reference)
import math

import jax
import jax.numpy as jnp
from jax import lax
from jax.experimental import pallas as pl
from jax.experimental.pallas import tpu as pltpu
from jax.experimental.pallas import tpu_sc as plsc

N = 10000
E = 320000
R = 4
D = 128
DE = 16
EPS = 1e-5
INV = 1.0 / math.sqrt(1.0 + EPS)

C = 4
CW = D // C
NR = N * R
ER = E // 128
G4 = 4
NGROUPS = ER // G4
NSUB = 16
SEG_ROWS_PER_TILE = NR // NSUB
GBASE = NGROUPS // NSUB
GREM = NGROUPS - NSUB * GBASE
NPAIRS = (GBASE + 1 + 1) // 2

BN_A = 2000
BE_B = 2560
RB = BE_B // 128


def _lrelu(x):
    return jnp.where(x >= 0, x, 0.1 * x)


def _node_mlp_body(hv, g1, b1, win, bin_, g2, b2, x4):
    x = _lrelu(hv[...] * INV * g1[...] + b1[...])
    x = jnp.dot(x, win[...], preferred_element_type=jnp.float32) + bin_[...]
    x = _lrelu(x * INV * g2[...] + b2[...])
    for c in range(C):
        x4[c] = x[:, c * CW:(c + 1) * CW]


def _node_mlp(h_v, g1, b1, Win, bIn, g2, b2):
    full = lambda shape: pl.BlockSpec(shape, lambda i: (0,) * len(shape))
    return pl.pallas_call(
        _node_mlp_body,
        grid=(N // BN_A,),
        in_specs=[
            pl.BlockSpec((BN_A, D), lambda i: (i, 0)),
            full((1, D)), full((1, D)), full((D, D)), full((1, D)),
            full((1, D)), full((1, D)),
        ],
        out_specs=pl.BlockSpec((C, BN_A, CW), lambda i: (0, i, 0)),
        out_shape=jax.ShapeDtypeStruct((C, N, CW), jnp.float32),
    )(h_v, g1.reshape(1, D), b1.reshape(1, D), Win, bIn.reshape(1, D),
      g2.reshape(1, D), b2.reshape(1, D))


def _seg_body(no, rl, seg):
    seg[...] = no[...] * R + rl[...]


def _seg_kernel(no2, rel2):
    return pl.pallas_call(
        _seg_body,
        grid=(E // BE_B,),
        in_specs=[
            pl.BlockSpec((1, RB, 128), lambda i: (i, 0, 0)),
            pl.BlockSpec((1, RB, 128), lambda i: (i, 0, 0)),
        ],
        out_specs=pl.BlockSpec((1, RB, 128), lambda i: (i, 0, 0)),
        out_shape=jax.ShapeDtypeStruct((E // BE_B, RB, 128), jnp.int32),
    )(no2, rel2)


def _edge_mlp_body(he, ge1, be1, we, be_, ge2, be2, eo):
    e = _lrelu(he[...] * INV * ge1[...] + be1[...])
    e = jnp.dot(e, we[...], preferred_element_type=jnp.float32) + be_[...]
    eo[...] = _lrelu(e * INV * ge2[...] + be2[...])


def _edge_mlp(h_e, ge1, be1, We, bE, ge2, be2):
    full = lambda shape: pl.BlockSpec(shape, lambda i: (0,) * len(shape))
    return pl.pallas_call(
        _edge_mlp_body,
        grid=(E // BE_B,),
        in_specs=[
            pl.BlockSpec((BE_B, DE), lambda i: (i, 0)),
            full((1, DE)), full((1, DE)), full((DE, D)), full((1, D)),
            full((1, D)), full((1, D)),
        ],
        out_specs=pl.BlockSpec((BE_B, D), lambda i: (i, 0)),
        out_shape=jax.ShapeDtypeStruct((E, D), jnp.float32),
    )(h_e, ge1.reshape(1, DE), be1.reshape(1, DE), We, bE.reshape(1, D),
      ge2.reshape(1, D), be2.reshape(1, D))


def _sc_prologue(sub, e_v, acc, seg_base):
    def _zb(i, carry):
        z = jnp.zeros((16,), jnp.float32)
        e_v[0, i, pl.ds(0, 16)] = z
        e_v[0, i, pl.ds(16, 16)] = z
        return carry
    lax.fori_loop(0, 500, _zb, 0)
    for k in range(SEG_ROWS_PER_TILE // 500):
        pltpu.sync_copy(e_v.at[0].at[pl.ds(0, 500)],
                        acc.at[pl.ds(seg_base + k * 500, 500)])
    plsc.subcore_barrier()


def _edge_scatter_x_body(x4, ni2, seg2, out, acc, ni_v, seg_v, xg_v,
                         sem_ld, sem_g, sem_sc):
    core = lax.axis_index("c")
    sub = lax.axis_index("s")
    seg_base = sub * SEG_ROWS_PER_TILE
    g_start = sub * GBASE + jnp.minimum(sub, GREM)
    g_count = jnp.where(sub < GREM, GBASE + 1, GBASE)

    for cc in range(2):
        chunk = core * 2 + cc
        _sc_prologue(sub, xg_v, acc, seg_base)

        def _loads(g, b, issue):
            row = (g_start + g) * G4
            f = pltpu.async_copy if issue else (
                lambda s, d, m: pltpu.make_async_copy(s, d, m).wait())
            f(ni2.at[pl.ds(row, G4)], ni_v.at[b], sem_ld)
            f(seg2.at[pl.ds(row, G4)], seg_v.at[b], sem_ld)

        def _wait_sc(b):
            for j in range(G4):
                pltpu.make_async_copy(
                    xg_v.at[b].at[pl.ds(j * 128, 128)],
                    acc.at[seg_v.at[b].at[j]], sem_sc).wait()

        _loads(0, 0, True)

        def _pair(pi, carry):
            for b in range(2):
                g = 2 * pi + b

                @pl.when(g < g_count)
                def _do_group():
                    _loads(g, b, False)

                    @pl.when(g >= 1)
                    def _wait_prev_scatters():
                        _wait_sc(1 - b)

                    @pl.when(g + 1 < g_count)
                    def _prefetch_next():
                        _loads(g + 1, 1 - b, True)

                    gds = [
                        pltpu.async_copy(
                            x4.at[chunk].at[ni_v.at[b].at[j]],
                            xg_v.at[b].at[pl.ds(j * 128, 128)], sem_g)
                        for j in range(G4)
                    ]
                    for dd in gds:
                        dd.wait()
                    for j in range(G4):
                        pltpu.async_copy(
                            xg_v.at[b].at[pl.ds(j * 128, 128)],
                            acc.at[seg_v.at[b].at[j]], sem_sc, add=True)
            return carry
        lax.fori_loop(0, NPAIRS, _pair, 0)
        last = g_count - 1

        @pl.when(last % 2 == 0)
        def _last_even():
            _wait_sc(0)

        @pl.when(last % 2 == 1)
        def _last_odd():
            _wait_sc(1)
        plsc.subcore_barrier()
        pltpu.sync_copy(
            acc.at[pl.ds(seg_base, SEG_ROWS_PER_TILE)],
            out.at[chunk].at[pl.ds(seg_base, SEG_ROWS_PER_TILE)])
        plsc.subcore_barrier()


def _edge_scatter_x(x4, ni2, seg2):
    mesh = plsc.VectorSubcoreMesh(core_axis_name="c", subcore_axis_name="s")
    return pl.kernel(
        _edge_scatter_x_body,
        out_type=jax.ShapeDtypeStruct((C, NR, CW), jnp.float32),
        mesh=mesh,
        compiler_params=pltpu.CompilerParams(use_tc_tiling_on_sc=False),
        scratch_types=[
            pltpu.VMEM_SHARED((NR, CW), jnp.float32),
            pltpu.VMEM((2, G4, 128), jnp.int32),
            pltpu.VMEM((2, G4, 128), jnp.int32),
            pltpu.VMEM((2, G4 * 128, CW), jnp.float32),
            pltpu.SemaphoreType.DMA,
            pltpu.SemaphoreType.DMA,
            pltpu.SemaphoreType.DMA,
        ],
    )(x4, ni2, seg2)


def _edge_scatter_e_body(e4, seg2, out, acc, seg_v, e_v,
                         sem_ld, sem_sc):
    core = lax.axis_index("c")
    sub = lax.axis_index("s")
    seg_base = sub * SEG_ROWS_PER_TILE
    g_start = sub * GBASE + jnp.minimum(sub, GREM)
    g_count = jnp.where(sub < GREM, GBASE + 1, GBASE)

    for cc in range(2):
        chunk = core * 2 + cc
        _sc_prologue(sub, e_v, acc, seg_base)

        def _loads(g, b, issue):
            row = (g_start + g) * G4
            f = pltpu.async_copy if issue else (
                lambda s, d, m: pltpu.make_async_copy(s, d, m).wait())
            f(seg2.at[pl.ds(row, G4)], seg_v.at[b], sem_ld)
            f(e4.at[pl.ds(row * 128, G4 * 128), pl.ds(chunk * CW, CW)],
              e_v.at[b], sem_ld)

        def _wait_sc(b):
            for j in range(G4):
                pltpu.make_async_copy(
                    e_v.at[b].at[pl.ds(j * 128, 128)],
                    acc.at[seg_v.at[b].at[j]], sem_sc).wait()

        _loads(0, 0, True)

        def _pair(pi, carry):
            for b in range(2):
                g = 2 * pi + b

                @pl.when(g < g_count)
                def _do_group():
                    _loads(g, b, False)

                    @pl.when(g >= 1)
                    def _wait_prev_scatters():
                        _wait_sc(1 - b)

                    @pl.when(g + 1 < g_count)
                    def _prefetch_next():
                        _loads(g + 1, 1 - b, True)

                    for j in range(G4):
                        pltpu.async_copy(
                            e_v.at[b].at[pl.ds(j * 128, 128)],
                            acc.at[seg_v.at[b].at[j]], sem_sc, add=True)
            return carry
        lax.fori_loop(0, NPAIRS, _pair, 0)
        last = g_count - 1

        @pl.when(last % 2 == 0)
        def _last_even():
            _wait_sc(0)

        @pl.when(last % 2 == 1)
        def _last_odd():
            _wait_sc(1)
        plsc.subcore_barrier()
        pltpu.sync_copy(
            acc.at[pl.ds(seg_base, SEG_ROWS_PER_TILE)],
            out.at[chunk].at[pl.ds(seg_base, SEG_ROWS_PER_TILE)])
        plsc.subcore_barrier()


def _edge_scatter_e(e4, seg2):
    mesh = plsc.VectorSubcoreMesh(core_axis_name="c", subcore_axis_name="s")
    return pl.kernel(
        _edge_scatter_e_body,
        out_type=jax.ShapeDtypeStruct((C, NR, CW), jnp.float32),
        mesh=mesh,
        compiler_params=pltpu.CompilerParams(use_tc_tiling_on_sc=False),
        scratch_types=[
            pltpu.VMEM_SHARED((NR, CW), jnp.float32),
            pltpu.VMEM((2, G4, 128), jnp.int32),
            pltpu.VMEM((2, G4 * 128, CW), jnp.float32),
            pltpu.SemaphoreType.DMA,
            pltpu.SemaphoreType.DMA,
        ],
    )(e4, seg2)


def _out_mlp_body(u4a, u4b, hv, glp, blp, wlp, go, bo, wo, out):
    u = jnp.zeros((BN_A, D), jnp.float32)
    for c in range(C):
        t = _lrelu((u4a[c] + u4b[c]) * INV * glp[c] + blp[c])
        u = u + jnp.dot(t, wlp[c], preferred_element_type=jnp.float32)
    o = _lrelu(u * INV * go[...] + bo[...])
    out[...] = jnp.dot(o, wo[...], preferred_element_type=jnp.float32) + hv[...]


def _out_mlp(U4a, U4b, h_v, glp, blp, Wlp, go, bo, Wo):
    full = lambda shape: pl.BlockSpec(shape, lambda i: (0,) * len(shape))
    return pl.pallas_call(
        _out_mlp_body,
        grid=(N // BN_A,),
        in_specs=[
            pl.BlockSpec((C, BN_A, D), lambda i: (0, i, 0)),
            pl.BlockSpec((C, BN_A, D), lambda i: (0, i, 0)),
            pl.BlockSpec((BN_A, D), lambda i: (i, 0)),
            full((C, 1, D)), full((C, 1, D)), full((C, D, D)),
            full((1, D)), full((1, D)), full((D, D)),
        ],
        out_specs=pl.BlockSpec((BN_A, D), lambda i: (i, 0)),
        out_shape=jax.ShapeDtypeStruct((N, D), jnp.float32),
    )(U4a, U4b, h_v, glp, blp, Wlp, go.reshape(1, D), bo.reshape(1, D), Wo)


def kernel(h_v, edge_index, h_e, g1, b1, Win, bIn, g2, b2, ge1, be1, We, bE,
           ge2, be2, gl, bl, Wl, go, bo, Wo):
    ni2 = edge_index[0].reshape(ER, 128)
    no2 = edge_index[1].reshape(E // BE_B, RB, 128)
    rel2 = edge_index[2].reshape(E // BE_B, RB, 128)

    x4 = _node_mlp(h_v, g1, b1, Win, bIn, g2, b2)
    seg3 = _seg_kernel(no2, rel2)
    seg2 = seg3.reshape(ER, 128)

    accX = _edge_scatter_x(x4, ni2, seg2)

    e4 = _edge_mlp(h_e, ge1, be1, We, bE, ge2, be2)
    accE = _edge_scatter_e(e4, seg2)

    U4a = accX.reshape(C, N, D)
    U4b = accE.reshape(C, N, D)

    q = jnp.arange(D)
    cidx = jnp.arange(C)
    perm = (q[None, :] // CW) * D + cidx[:, None] * CW + q[None, :] % CW
    perm = perm.reshape(-1)
    Wlp = Wl[perm].reshape(C, D, D)
    glp = gl[perm].reshape(C, 1, D)
    blp = bl[perm].reshape(C, 1, D)

    return _out_mlp(U4a, U4b, h_v, glp, blp, Wlp, go, bo, Wo)


# speedup vs baseline: 1.0251x; 1.0251x over previous
import math

import jax
import jax.numpy as jnp
from jax import lax
from jax.experimental import pallas as pl
from jax.experimental.pallas import tpu as pltpu
from jax.experimental.pallas import tpu_sc as plsc

N = 10000
E = 320000
R = 4
D = 128
DE = 16
EPS = 1e-5
INV = 1.0 / math.sqrt(1.0 + EPS)

C = 4
CW = D // C
NR = N * R
G4 = 4
NSUB = 16
SEG_ROWS_PER_TILE = NR // NSUB

BN_A = 2000
BE_B = 2560

E_A = 163840
E_B = E - E_A


def _lrelu(x):
    return jnp.where(x >= 0, x, 0.1 * x)


def _node_mlp_body(hv, g1, b1, win, bin_, g2, b2, x4):
    x = _lrelu(hv[...] * INV * g1[...] + b1[...])
    x = jnp.dot(x, win[...], preferred_element_type=jnp.float32) + bin_[...]
    x = _lrelu(x * INV * g2[...] + b2[...])
    for c in range(C):
        x4[c] = x[:, c * CW:(c + 1) * CW]


def _node_mlp(h_v, g1, b1, Win, bIn, g2, b2):
    grid = (N // BN_A,)
    full = lambda shape: pl.BlockSpec(shape, lambda i: (0,) * len(shape))
    return pl.pallas_call(
        _node_mlp_body,
        grid=grid,
        in_specs=[
            pl.BlockSpec((BN_A, D), lambda i: (i, 0)),
            full((1, D)), full((1, D)), full((D, D)), full((1, D)),
            full((1, D)), full((1, D)),
        ],
        out_specs=pl.BlockSpec((C, BN_A, CW), lambda i: (0, i, 0)),
        out_shape=jax.ShapeDtypeStruct((C, N, CW), jnp.float32),
    )(h_v, g1.reshape(1, D), b1.reshape(1, D), Win, bIn.reshape(1, D),
      g2.reshape(1, D), b2.reshape(1, D))


def _edge_mlp_body(he, ge1, be1, we, be_, ge2, be2, no, rl, eo, seg):
    e = _lrelu(he[...] * INV * ge1[...] + be1[...])
    e = jnp.dot(e, we[...], preferred_element_type=jnp.float32) + be_[...]
    eo[...] = _lrelu(e * INV * ge2[...] + be2[...])
    seg[...] = no[...] * R + rl[...]


def _edge_mlp(h_e, ge1, be1, We, bE, ge2, be2, no2, rel2, eh):
    grid = (eh // BE_B,)
    rows = BE_B // 128
    full = lambda shape: pl.BlockSpec(shape, lambda i: (0,) * len(shape))
    return pl.pallas_call(
        _edge_mlp_body,
        grid=grid,
        in_specs=[
            pl.BlockSpec((BE_B, DE), lambda i: (i, 0)),
            full((1, DE)), full((1, DE)), full((DE, D)), full((1, D)),
            full((1, D)), full((1, D)),
            pl.BlockSpec((1, rows, 128), lambda i: (i, 0, 0)),
            pl.BlockSpec((1, rows, 128), lambda i: (i, 0, 0)),
        ],
        out_specs=[
            pl.BlockSpec((BE_B, D), lambda i: (i, 0)),
            pl.BlockSpec((1, rows, 128), lambda i: (i, 0, 0)),
        ],
        out_shape=[
            jax.ShapeDtypeStruct((eh, D), jnp.float32),
            jax.ShapeDtypeStruct((eh // BE_B, rows, 128), jnp.int32),
        ],
    )(h_e, ge1.reshape(1, DE), be1.reshape(1, DE), We, bE.reshape(1, D),
      ge2.reshape(1, D), be2.reshape(1, D), no2, rel2)


def _make_edge_scatter_body(ngroups):
    base = ngroups // NSUB
    rem = ngroups - NSUB * base
    maxcount = base + (1 if rem else 0)
    npairs = (maxcount + 1) // 2

    def _edge_scatter_body(x4, e4, ni2, seg2, out, acc, ni_v, seg_v, e_v,
                           xg_v, sem_ld, sem_g, sem_sc):
        core = lax.axis_index("c")
        sub = lax.axis_index("s")

        seg_base = sub * SEG_ROWS_PER_TILE
        g_start = sub * base + jnp.minimum(sub, rem)
        g_count = jnp.where(sub < rem, base + 1, base)

        for cc in range(2):
            chunk = core * 2 + cc
            def _zb(i, carry):
                z = jnp.zeros((16,), jnp.float32)
                e_v[0, i, pl.ds(0, 16)] = z
                e_v[0, i, pl.ds(16, 16)] = z
                return carry
            lax.fori_loop(0, 500, _zb, 0)
            for k in range(SEG_ROWS_PER_TILE // 500):
                pltpu.sync_copy(e_v.at[0].at[pl.ds(0, 500)],
                                acc.at[pl.ds(seg_base + k * 500, 500)])
            plsc.subcore_barrier()

            def _loads(g, b, issue):
                row = (g_start + g) * G4
                f = pltpu.async_copy if issue else (
                    lambda s, d, m: pltpu.make_async_copy(s, d, m).wait())
                f(seg2.at[pl.ds(row, G4)], seg_v.at[b], sem_ld)
                f(e4.at[pl.ds(row * 128, G4 * 128), pl.ds(chunk * CW, CW)],
                  e_v.at[b], sem_ld)

            def _load_ni(g, issue):
                row = (g_start + g) * G4
                f = pltpu.async_copy if issue else (
                    lambda s, d, m: pltpu.make_async_copy(s, d, m).wait())
                f(ni2.at[pl.ds(row, G4)], ni_v, sem_ld)

            def _wait_sc(b):
                for j in range(G4):
                    pltpu.make_async_copy(e_v.at[b].at[pl.ds(j * 128, 128)],
                                          acc.at[seg_v.at[b].at[j]],
                                          sem_sc).wait()
                for j in range(G4):
                    pltpu.make_async_copy(xg_v.at[pl.ds(j * 128, 128)],
                                          acc.at[seg_v.at[b].at[j]],
                                          sem_sc).wait()

            _loads(0, 0, True)
            _load_ni(0, True)

            def _pair(pi, carry):
                for b in range(2):
                    g = 2 * pi + b

                    @pl.when(g < g_count)
                    def _do_group():
                        _loads(g, b, False)
                        _load_ni(g, False)

                        @pl.when(g >= 1)
                        def _wait_prev_scatters():
                            _wait_sc(1 - b)

                        @pl.when(g + 1 < g_count)
                        def _prefetch_next():
                            _loads(g + 1, 1 - b, True)

                        gds = [
                            pltpu.async_copy(
                                x4.at[chunk].at[ni_v.at[j]],
                                xg_v.at[pl.ds(j * 128, 128)], sem_g)
                            for j in range(G4)
                        ]
                        for j in range(G4):
                            pltpu.async_copy(
                                e_v.at[b].at[pl.ds(j * 128, 128)],
                                acc.at[seg_v.at[b].at[j]], sem_sc, add=True)
                        for dd in gds:
                            dd.wait()

                        @pl.when(g + 1 < g_count)
                        def _prefetch_next_ni():
                            _load_ni(g + 1, True)

                        for j in range(G4):
                            pltpu.async_copy(
                                xg_v.at[pl.ds(j * 128, 128)],
                                acc.at[seg_v.at[b].at[j]], sem_sc, add=True)
                return carry
            lax.fori_loop(0, npairs, _pair, 0)
            last = g_count - 1

            @pl.when(last % 2 == 0)
            def _last_even():
                _wait_sc(0)

            @pl.when(last % 2 == 1)
            def _last_odd():
                _wait_sc(1)
            plsc.subcore_barrier()
            pltpu.sync_copy(
                acc.at[pl.ds(seg_base, SEG_ROWS_PER_TILE)],
                out.at[chunk].at[pl.ds(seg_base, SEG_ROWS_PER_TILE)])
            plsc.subcore_barrier()

    return _edge_scatter_body


def _edge_scatter(x4, e4, ni2, seg2, eh):
    er = eh // 128
    mesh = plsc.VectorSubcoreMesh(core_axis_name="c", subcore_axis_name="s")
    return pl.kernel(
        _make_edge_scatter_body(er // G4),
        out_type=jax.ShapeDtypeStruct((C, NR, CW), jnp.float32),
        mesh=mesh,
        compiler_params=pltpu.CompilerParams(use_tc_tiling_on_sc=False),
        scratch_types=[
            pltpu.VMEM_SHARED((NR, CW), jnp.float32),
            pltpu.VMEM((G4, 128), jnp.int32),
            pltpu.VMEM((2, G4, 128), jnp.int32),
            pltpu.VMEM((2, G4 * 128, CW), jnp.float32),
            pltpu.VMEM((G4 * 128, CW), jnp.float32),
            pltpu.SemaphoreType.DMA,
            pltpu.SemaphoreType.DMA,
            pltpu.SemaphoreType.DMA,
        ],
    )(x4, e4, ni2, seg2)


def _out_mlp_body(u4a, u4b, hv, glp, blp, wlp, go, bo, wo, out):
    u = jnp.zeros((BN_A, D), jnp.float32)
    for c in range(C):
        t = _lrelu((u4a[c] + u4b[c]) * INV * glp[c] + blp[c])
        u = u + jnp.dot(t, wlp[c], preferred_element_type=jnp.float32)
    o = _lrelu(u * INV * go[...] + bo[...])
    out[...] = jnp.dot(o, wo[...], preferred_element_type=jnp.float32) + hv[...]


def _out_mlp(U4a, U4b, h_v, glp, blp, Wlp, go, bo, Wo):
    grid = (N // BN_A,)
    full = lambda shape: pl.BlockSpec(shape, lambda i: (0,) * len(shape))
    return pl.pallas_call(
        _out_mlp_body,
        grid=grid,
        in_specs=[
            pl.BlockSpec((C, BN_A, D), lambda i: (0, i, 0)),
            pl.BlockSpec((C, BN_A, D), lambda i: (0, i, 0)),
            pl.BlockSpec((BN_A, D), lambda i: (i, 0)),
            full((C, 1, D)), full((C, 1, D)), full((C, D, D)),
            full((1, D)), full((1, D)), full((D, D)),
        ],
        out_specs=pl.BlockSpec((BN_A, D), lambda i: (i, 0)),
        out_shape=jax.ShapeDtypeStruct((N, D), jnp.float32),
    )(U4a, U4b, h_v, glp, blp, Wlp, go.reshape(1, D), bo.reshape(1, D), Wo)


def kernel(h_v, edge_index, h_e, g1, b1, Win, bIn, g2, b2, ge1, be1, We, bE,
           ge2, be2, gl, bl, Wl, go, bo, Wo):
    rows_b = BE_B // 128

    ni = edge_index[0]
    no = edge_index[1]
    rl = edge_index[2]

    x4 = _node_mlp(h_v, g1, b1, Win, bIn, g2, b2)

    halves = []
    for lo, eh in ((0, E_A), (E_A, E_B)):
        no2 = lax.dynamic_slice_in_dim(no, lo, eh).reshape(
            eh // BE_B, rows_b, 128)
        rel2 = lax.dynamic_slice_in_dim(rl, lo, eh).reshape(
            eh // BE_B, rows_b, 128)
        heh = lax.dynamic_slice_in_dim(h_e, lo, eh)
        ni2 = lax.dynamic_slice_in_dim(ni, lo, eh).reshape(eh // 128, 128)
        halves.append((ni2, no2, rel2, heh, eh))

    accs = []
    for ni2, no2, rel2, heh, eh in halves:
        e4, seg3 = _edge_mlp(heh, ge1, be1, We, bE, ge2, be2, no2, rel2, eh)
        accs.append(
            _edge_scatter(x4, e4, ni2, seg3.reshape(eh // 128, 128), eh))

    U4a = accs[0].reshape(C, N, D)
    U4b = accs[1].reshape(C, N, D)

    q = jnp.arange(D)
    cidx = jnp.arange(C)
    perm = (q[None, :] // CW) * D + cidx[:, None] * CW + q[None, :] % CW
    perm = perm.reshape(-1)
    Wlp = Wl[perm].reshape(C, D, D)
    glp = gl[perm].reshape(C, 1, D)
    blp = bl[perm].reshape(C, 1, D)

    return _out_mlp(U4a, U4b, h_v, glp, blp, Wlp, go, bo, Wo)
